# Initial kernel scaffold; baseline (speedup 1.0000x reference)
import jax
import jax.numpy as jnp
from jax.experimental import pallas as pl


def kernel(x, edge_index, W1, b1, W2, b2):
    raise NotImplementedError("write your pallas kernel here")


# speedup vs baseline: 11.7124x; 11.7124x over previous
import functools

import jax
import jax.numpy as jnp
from jax import lax
from jax.experimental import pallas as pl
from jax.experimental.pallas import tpu as pltpu
from jax.experimental.pallas import tpu_sc as plsc

NC = 2
NS = 16
NW = NC * NS
K = 128
DEGW = 128


def _sc_mesh():
    return plsc.VectorSubcoreMesh(core_axis_name="c", subcore_axis_name="s")


def _make_sc_deg(C, n_pad, rows_pt):
    @functools.partial(
        pl.kernel,
        out_type=jax.ShapeDtypeStruct((NC, n_pad, DEGW), jnp.float32),
        mesh=_sc_mesh(),
        scratch_types=[
            pltpu.MemorySpace.VMEM((C, K), jnp.int32),
            pltpu.MemorySpace.VMEM((K, DEGW), jnp.float32),
            pltpu.MemorySpace.VMEM_SHARED((n_pad, DEGW), jnp.float32),
            pltpu.SemaphoreType.DMA,
        ],
    )
    def deg_kernel(dst_hbm, ones_hbm, zeros_hbm, out_hbm, idx_v, ones_v, acc, sem):
        c = lax.axis_index("c")
        s = lax.axis_index("s")
        wid = s * NC + c
        pltpu.sync_copy(dst_hbm.at[wid], idx_v)
        pltpu.sync_copy(ones_hbm, ones_v)
        pltpu.sync_copy(zeros_hbm, acc.at[pl.ds(s * rows_pt, rows_pt)])
        plsc.subcore_barrier()

        def body(j, carry):
            pltpu.sync_copy(ones_v, acc.at[idx_v.at[j]], add=True)
            return carry

        lax.fori_loop(0, C, body, 0)
        plsc.subcore_barrier()
        pltpu.sync_copy(
            acc.at[pl.ds(s * rows_pt, rows_pt)],
            out_hbm.at[c, pl.ds(s * rows_pt, rows_pt)],
        )

    return deg_kernel


def _make_sc_edges(C, n_pad, rows_pt, d):
    @functools.partial(
        pl.kernel,
        out_type=jax.ShapeDtypeStruct((NC, n_pad, d), jnp.float32),
        mesh=_sc_mesh(),
        scratch_types=[
            pltpu.MemorySpace.VMEM((C, K), jnp.int32),
            pltpu.MemorySpace.VMEM((C, K), jnp.int32),
            pltpu.MemorySpace.VMEM((K, d), jnp.float32),
            pltpu.MemorySpace.VMEM_SHARED((n_pad, d), jnp.float32),
            pltpu.SemaphoreType.DMA,
        ],
    )
    def edge_kernel(hp_hbm, src_hbm, dst_hbm, zeros_hbm, out_hbm,
                    src_v, dst_v, buf, acc, sem):
        c = lax.axis_index("c")
        s = lax.axis_index("s")
        wid = s * NC + c
        pltpu.sync_copy(src_hbm.at[wid], src_v)
        pltpu.sync_copy(dst_hbm.at[wid], dst_v)
        pltpu.sync_copy(zeros_hbm, acc.at[pl.ds(s * rows_pt, rows_pt)])
        plsc.subcore_barrier()

        def body(j, carry):
            pltpu.async_copy(hp_hbm.at[src_v.at[j]], buf, sem).wait()
            pltpu.sync_copy(buf, acc.at[dst_v.at[j]], add=True)
            return carry

        lax.fori_loop(0, C, body, 0)
        plsc.subcore_barrier()
        pltpu.sync_copy(
            acc.at[pl.ds(s * rows_pt, rows_pt)],
            out_hbm.at[c, pl.ds(s * rows_pt, rows_pt)],
        )

    return edge_kernel


def _tc_call(fn, out_shape):
    return pl.pallas_call(fn, out_shape=out_shape)


def _make_tc1(n, n_pad, din, d):
    def body(x_ref, w_ref, cnt_ref, hp_ref):
        deg = 1.0 + cnt_ref[0, 0:n, 0:1] + cnt_ref[1, 0:n, 0:1]
        dis = lax.rsqrt(deg)
        h = jnp.dot(x_ref[...], w_ref[...], preferred_element_type=jnp.float32)
        hp_ref[...] = h * dis

    return _tc_call(body, jax.ShapeDtypeStruct((n, d), jnp.float32))


def _make_tc2(n, n_pad, d, dout):
    def body(acc_ref, hp_ref, cnt_ref, w_ref, b_ref, hp2_ref):
        deg = 1.0 + cnt_ref[0, 0:n, 0:1] + cnt_ref[1, 0:n, 0:1]
        dis = lax.rsqrt(deg)
        t = (acc_ref[0, 0:n, :] + acc_ref[1, 0:n, :] + hp_ref[...]) * dis + b_ref[...]
        y = jnp.maximum(t, 0.0)
        h2 = jnp.dot(y, w_ref[...], preferred_element_type=jnp.float32)
        hp2_ref[...] = h2 * dis

    return _tc_call(body, jax.ShapeDtypeStruct((n, dout), jnp.float32))


def _make_tc3(n, n_pad, d):
    def body(acc_ref, hp_ref, cnt_ref, b_ref, out_ref):
        deg = 1.0 + cnt_ref[0, 0:n, 0:1] + cnt_ref[1, 0:n, 0:1]
        dis = lax.rsqrt(deg)
        out_ref[...] = (
            acc_ref[0, 0:n, :] + acc_ref[1, 0:n, :] + hp_ref[...]
        ) * dis + b_ref[...]

    return _tc_call(body, jax.ShapeDtypeStruct((n, d), jnp.float32))


def kernel(x, edge_index, W1, b1, W2, b2):
    n, din = x.shape
    dh = W1.shape[1]
    dout = W2.shape[1]
    e = edge_index.shape[1]

    C = -(-e // (NW * K))
    e_pad = NW * C * K
    rows_pt = -(-(n + 1) // (NS * 8)) * 8
    n_pad = rows_pt * NS

    src = edge_index[0]
    dst = edge_index[1]
    pad = e_pad - e
    src_r = jnp.concatenate([src, jnp.zeros((pad,), jnp.int32)]).reshape(NW, C, K)
    dst_r = jnp.concatenate([dst, jnp.full((pad,), n, jnp.int32)]).reshape(NW, C, K)

    ones8 = jnp.ones((K, DEGW), jnp.float32)
    zerosd = jnp.zeros((rows_pt, dh), jnp.float32)
    zeros8 = zerosd if DEGW == dh else jnp.zeros((rows_pt, DEGW), jnp.float32)
    b1r = b1.reshape(1, dh)
    b2r = b2.reshape(1, dout)

    sc_deg = _make_sc_deg(C, n_pad, rows_pt)
    sc_edges = _make_sc_edges(C, n_pad, rows_pt, dh)
    tc1 = _make_tc1(n, n_pad, din, dh)
    tc2 = _make_tc2(n, n_pad, dh, dout)
    tc3 = _make_tc3(n, n_pad, dh)

    cnt = sc_deg(dst_r, ones8, zeros8)
    hp1 = tc1(x, W1, cnt)
    acc1 = sc_edges(hp1, src_r, dst_r, zerosd)
    hp2 = tc2(acc1, hp1, cnt, W2, b1r)
    acc2 = sc_edges(hp2, src_r, dst_r, zerosd)
    out = tc3(acc2, hp2, cnt, b2r)
    return out
